# baseline (speedup 1.0000x reference)
import functools

import jax
import jax.numpy as jnp
from jax import lax
from jax.experimental import pallas as pl
from jax.experimental.pallas import tpu as pltpu
from jax.experimental.pallas import tpu_sc as plsc

B = 4096
D = 32
L = 8

_NC, _NS = 2, 16
_NW = _NC * _NS
_RW = B // _NW
_TK = _RW * L
_CH = 8
_NCH = _RW // _CH

_POOL_ROW_STARTS = (32, 131, 166, 199, 237)
_BIG_ROW_STARTS = (0, 64, 96)
_NUM_ROWS = (128, 129, 130, 163, 164, 165, 198, 231, 232, 233, 234, 235, 236)


def _make_sc_gather():
    mesh = plsc.VectorSubcoreMesh(
        core_axis_name="c", subcore_axis_name="s",
        num_cores=_NC, num_subcores=_NS)

    @functools.partial(
        pl.kernel,
        out_type=(
            jax.ShapeDtypeStruct((5, B, D), jnp.float32),
            jax.ShapeDtypeStruct((3, B, D), jnp.float32),
            jax.ShapeDtypeStruct((B,), jnp.float32),
        ),
        mesh=mesh,
        scratch_types=[
            pltpu.VMEM((144,), jnp.int32),
            pltpu.VMEM((_TK,), jnp.int32),
            pltpu.VMEM((_TK,), jnp.int32),
            pltpu.VMEM((_TK, D), jnp.float32),
            pltpu.VMEM((2, _CH, D, 128), jnp.float32),
            pltpu.VMEM((_CH, D), jnp.float32),
            pltpu.VMEM((_RW, D), jnp.float32),
            pltpu.VMEM((_RW,), jnp.float32),
            pltpu.VMEM((_TK,), jnp.int32),
            pltpu.VMEM((_TK,), jnp.float32),
            pltpu.VMEM_SHARED((_NS * _RW, D), jnp.float32),
            pltpu.VMEM_SHARED((_NS * _RW, D), jnp.float32),
            pltpu.VMEM_SHARED((_NS * _RW,), jnp.float32),
            pltpu.SemaphoreType.DMA,
            pltpu.SemaphoreType.DMA,
            pltpu.SemaphoreType.DMA,
            pltpu.SemaphoreType.DMA,
        ],
        compiler_params=pltpu.CompilerParams(
            needs_layout_passes=False, use_tc_tiling_on_sc=False),
    )
    def sc_gather(track_i, artist_i, album_i, name_i, genres_i, key_i,
                  mode_i, ts_i,
                  t_track, t_artist, t_album, t_name, t_genres, t_key,
                  t_mode, t_ts,
                  ea, eb, n0, idx_pad, idx_l, idx_l2, rows_l, blk, pool8,
                  zsrc, zn, pidx, zbuf, acc, acc2, n0_acc,
                  semA, semB, semg, semg2):
        sid = lax.axis_index("s")
        wid = sid * _NC + lax.axis_index("c")
        base = wid * _RW
        tb = sid * _RW
        c_lo = lax.iota(jnp.int32, 16)
        sems = (semA, semB)

        pltpu.sync_copy(name_i.at[pl.ds(base * L, _TK)], idx_l)
        name_gather = pltpu.async_copy(t_name.at[idx_l], rows_l, semg)

        def pidx_body(j, carry):
            pidx[pl.ds(j * 16, 16)] = tb + ((c_lo + j * 16) >> 3)
            return carry

        lax.fori_loop(0, _TK // 16, pidx_body, 0)

        def zero_body(j, carry):
            zsrc[j, pl.ds(0, 16)] = jnp.zeros((16,), jnp.float32)
            zsrc[j, pl.ds(16, 16)] = jnp.zeros((16,), jnp.float32)
            return carry

        lax.fori_loop(0, _RW, zero_body, 0)

        def zn_body(j, carry):
            zn[pl.ds(j * 16, 16)] = jnp.zeros((16,), jnp.float32)
            return carry

        lax.fori_loop(0, _RW // 16, zn_body, 0)

        pltpu.sync_copy(zsrc, acc.at[pl.ds(tb, _RW)])
        pltpu.sync_copy(zsrc, acc2.at[pl.ds(tb, _RW)])
        pltpu.sync_copy(zn, n0_acc.at[pl.ds(tb, _RW)])

        def fire(th, ch, b):
            iv = idx_pad[pl.ds(ch * _CH, 16)]
            for k in range(_CH):
                joff = pl.multiple_of((iv[k] >> 7) * 128, 128)
                pltpu.async_copy(
                    th.at[:, pl.ds(joff, 128)], blk.at[b, k], sems[b])

        def drain(th, b):
            for k in range(_CH):
                pltpu.make_async_copy(
                    th.at[:, pl.ds(0, 128)], blk.at[b, k], sems[b]).wait()

        def extract(f, ch, b):
            iv = idx_pad[pl.ds(ch * _CH, 16)]
            bb = jnp.full((16,), b, jnp.int32)
            for k in range(_CH):
                col = jnp.full((16,), iv[k] & 127, jnp.int32)
                kk = jnp.full((16,), k, jnp.int32)
                pool8[k, pl.ds(0, 16)] = plsc.load_gather(
                    blk, [bb, kk, c_lo, col])
                pool8[k, pl.ds(16, 16)] = plsc.load_gather(
                    blk, [bb, kk, c_lo + 16, col])
            pltpu.sync_copy(
                pool8, eb.at[f, pl.ds(base + ch * _CH, _CH), :])

        def big_feature(f, ih, th):
            pltpu.sync_copy(ih.at[pl.ds(base, _RW)],
                            idx_pad.at[pl.ds(0, _RW)])
            fire(th, 0, 0)
            fire(th, 1, 1)

            def pair_body(j, carry):
                for b in range(2):
                    ch = j * 2 + b
                    drain(th, b)
                    extract(f, ch, b)
                    fire(th, ch + 2, b)
                return carry

            lax.fori_loop(0, _NCH // 2 - 1, pair_body, 0)
            for b in range(2):
                drain(th, b)
                extract(f, _NCH - 2 + b, b)

        big_feature(0, track_i, t_track)

        name_gather.wait()
        pltpu.sync_copy(rows_l, acc.at[pidx], add=True)
        pltpu.sync_copy(acc.at[pl.ds(tb, _RW)], ea.at[0, pl.ds(base, _RW), :])

        pltpu.sync_copy(genres_i.at[pl.ds(base * L, _TK)], idx_l2)
        gen_gather = pltpu.async_copy(t_genres.at[idx_l2], rows_l, semg2)

        def zmask_body(j, carry):
            zbuf[pl.ds(j * 16, 16)] = jnp.where(
                idx_l2[pl.ds(j * 16, 16)] == 0, jnp.float32(1.0),
                jnp.float32(0.0))
            return carry

        lax.fori_loop(0, _TK // 16, zmask_body, 0)

        big_feature(1, artist_i, t_artist)
        big_feature(2, album_i, t_album)

        gen_gather.wait()
        pltpu.sync_copy(rows_l, acc2.at[pidx], add=True)
        pltpu.sync_copy(zbuf, n0_acc.at[pidx], add=True)
        pltpu.sync_copy(acc2.at[pl.ds(tb, _RW)], ea.at[1, pl.ds(base, _RW), :])
        pltpu.sync_copy(n0_acc.at[pl.ds(tb, _RW)], n0.at[pl.ds(base, _RW)])

        for f, ih, th in ((2, key_i, t_key), (3, mode_i, t_mode),
                          (4, ts_i, t_ts)):
            pltpu.sync_copy(ih.at[pl.ds(base, _RW)],
                            idx_pad.at[pl.ds(0, _RW)])
            pltpu.async_copy(
                th.at[idx_pad.at[pl.ds(0, _RW)]], zsrc, semg).wait()
            pltpu.sync_copy(zsrc, ea.at[f, pl.ds(base, _RW), :])

    return sc_gather


_sc_gather_cached = functools.cache(_make_sc_gather)

_R = 512


def _mlp_body(ea_ref, eb_ref, n_ref, n0_ref, row0_ref, w1a_ref, w1b_ref,
              w1n_ref, b1_ref, w2_ref, b2_ref, w3_ref, b3_ref, g_ref,
              bt_ref, out_ref):
    f32 = jnp.float32
    h = jnp.dot(n_ref[...], w1n_ref[...], preferred_element_type=f32)
    h = h + jnp.dot(ea_ref[0], w1a_ref[0], preferred_element_type=f32)
    n0 = n0_ref[...]
    eg = (ea_ref[1] - n0 * row0_ref[...]) / jnp.maximum(8.0 - n0, 1.0)
    h = h + jnp.dot(eg, w1a_ref[1], preferred_element_type=f32)
    for f in range(2, 5):
        h = h + jnp.dot(ea_ref[f], w1a_ref[f], preferred_element_type=f32)
    for f in range(3):
        h = h + jnp.dot(eb_ref[f], w1b_ref[f], preferred_element_type=f32)
    h = jnp.maximum(h + b1_ref[...], 0.0)
    h = jnp.dot(h, w2_ref[...], preferred_element_type=f32) + b2_ref[...]
    h = jnp.maximum(h, 0.0)
    h = jnp.dot(h, w3_ref[...], preferred_element_type=f32) + b3_ref[...]
    mu = jnp.mean(h, axis=1, keepdims=True)
    d = h - mu
    var = jnp.mean(d * d, axis=1, keepdims=True)
    out_ref[...] = g_ref[...] * d / jnp.sqrt(var + 1e-3) + bt_ref[...]


def _mlp_call(ea, eb, nmat, n0, row0, w1a, w1b, w1n, b1, w2, b2, w3, b3,
              gamma, beta):
    full = lambda shape: pl.BlockSpec(shape, lambda i: (0, 0))
    return pl.pallas_call(
        _mlp_body,
        grid=(B // _R,),
        in_specs=[
            pl.BlockSpec((5, _R, D), lambda i: (0, i, 0)),
            pl.BlockSpec((3, _R, D), lambda i: (0, i, 0)),
            pl.BlockSpec((_R, 16), lambda i: (i, 0)),
            pl.BlockSpec((_R, 1), lambda i: (i, 0)),
            full((1, D)),
            pl.BlockSpec((5, D, 512), lambda i: (0, 0, 0)),
            pl.BlockSpec((3, D, 512), lambda i: (0, 0, 0)),
            full((16, 512)), full((1, 512)),
            full((512, 256)), full((1, 256)),
            full((256, 128)), full((1, 128)),
            full((1, 128)), full((1, 128)),
        ],
        out_specs=pl.BlockSpec((_R, 128), lambda i: (i, 0)),
        out_shape=jax.ShapeDtypeStruct((B, 128), jnp.float32),
    )(ea, eb, nmat, n0, row0, w1a, w1b, w1n, b1, w2, b2, w3, b3, gamma,
      beta)


def kernel(track_uri_can, track_name_can, artist_uri_can, album_uri_can,
           artist_genres_can, track_key_can, track_mode_can,
           time_signature_can, duration_ms_can, track_pop_can,
           artist_pop_can, artist_followers_can, track_danceability_can,
           track_energy_can, track_loudness_can, track_speechiness_can,
           track_acousticness_can, track_instrumentalness_can,
           track_liveness_can, track_valence_can, track_tempo_can,
           track_uri_table, track_name_table, artist_uri_table,
           album_uri_table, artist_genres_table, track_key_table,
           track_mode_table, time_signature_table, W1, b1, W2, b2, W3, b3,
           gamma, beta):
    i32 = jnp.int32
    ea, eb, n0 = _sc_gather_cached()(
        track_uri_can.astype(i32),
        artist_uri_can.astype(i32),
        album_uri_can.astype(i32),
        track_name_can.astype(i32).reshape(-1),
        artist_genres_can.astype(i32).reshape(-1),
        track_key_can.astype(i32),
        track_mode_can.astype(i32),
        time_signature_can.astype(i32),
        track_uri_table.T, artist_uri_table.T, album_uri_table.T,
        track_name_table, artist_genres_table, track_key_table,
        track_mode_table, time_signature_table)

    z = jnp.zeros_like(duration_ms_can)
    nmat = jnp.stack(
        [duration_ms_can, track_pop_can, artist_pop_can,
         artist_followers_can, track_danceability_can, track_energy_can,
         track_loudness_can, track_speechiness_can, track_acousticness_can,
         track_instrumentalness_can, track_liveness_can, track_valence_can,
         track_tempo_can, z, z, z], axis=1)

    w1a = jnp.stack(
        [W1[32:32 + D] * (1.0 / L)] +
        [W1[s:s + D] for s in _POOL_ROW_STARTS[1:]], axis=0)
    w1b = jnp.stack([W1[s:s + D] for s in _BIG_ROW_STARTS], axis=0)
    w1n = jnp.concatenate(
        [W1[jnp.array(_NUM_ROWS)], jnp.zeros((3, W1.shape[1]), W1.dtype)],
        axis=0)

    return _mlp_call(ea, eb, nmat, n0.reshape(-1, 1),
                     artist_genres_table[0:1], w1a, w1b, w1n,
                     b1.reshape(1, -1), W2, b2.reshape(1, -1), W3,
                     b3.reshape(1, -1), gamma.reshape(1, -1),
                     beta.reshape(1, -1))


# speedup vs baseline: 35.1102x; 35.1102x over previous
import functools

import jax
import jax.numpy as jnp
from jax import lax
from jax.experimental import pallas as pl
from jax.experimental.pallas import tpu as pltpu
from jax.experimental.pallas import tpu_sc as plsc

B = 4096
D = 32
L = 8

_NC, _NS = 2, 16
_NW = _NC * _NS
_RW = B // _NW
_TK = _RW * L
_CH = 8
_NCH = _RW // _CH

_POOL_ROW_STARTS = (32, 131, 166, 199, 237)
_BIG_ROW_STARTS = (0, 64, 96)
_NUM_ROWS = (128, 129, 130, 163, 164, 165, 198, 231, 232, 233, 234, 235, 236)


def _make_sc_pool():
    mesh = plsc.VectorSubcoreMesh(
        core_axis_name="c", subcore_axis_name="s",
        num_cores=_NC, num_subcores=_NS)

    @functools.partial(
        pl.kernel,
        out_type=jax.ShapeDtypeStruct((5, B, D), jnp.float32),
        mesh=mesh,
        scratch_types=[
            pltpu.VMEM((_RW,), jnp.int32),
            pltpu.VMEM((_TK,), jnp.int32),
            pltpu.VMEM((_TK,), jnp.int32),
            pltpu.VMEM((_TK, D), jnp.float32),
            pltpu.VMEM((_TK, D), jnp.float32),
            pltpu.VMEM((_RW, D), jnp.float32),
            pltpu.SemaphoreType.DMA,
            pltpu.SemaphoreType.DMA,
            pltpu.SemaphoreType.DMA,
        ],
        compiler_params=pltpu.CompilerParams(use_tc_tiling_on_sc=False),
    )
    def sc_pool(name_i, genres_i, key_i, mode_i, ts_i,
                t_name, t_genres, t_key, t_mode, t_ts,
                ea, idx_s, idx_l, idx_l2, rows_l, rows_l2, pool,
                sem, semg, semg2):
        wid = lax.axis_index("s") * _NC + lax.axis_index("c")
        base = wid * _RW

        for t in range(L):
            pltpu.sync_copy(name_i.at[t, pl.ds(base, _RW)],
                            idx_l.at[pl.ds(t * _RW, _RW)])
            pltpu.sync_copy(genres_i.at[t, pl.ds(base, _RW)],
                            idx_l2.at[pl.ds(t * _RW, _RW)])
        name_gather = pltpu.async_copy(t_name.at[idx_l], rows_l, semg)
        gen_gather = pltpu.async_copy(t_genres.at[idx_l2], rows_l2, semg2)

        for f, ih, th in ((2, key_i, t_key), (3, mode_i, t_mode),
                          (4, ts_i, t_ts)):
            pltpu.sync_copy(ih.at[pl.ds(base, _RW)], idx_s)
            pltpu.async_copy(th.at[idx_s], pool, sem).wait()
            pltpu.sync_copy(pool, ea.at[f, pl.ds(base, _RW), :])

        def reduce_rows(rows):
            def body(i4, carry):
                for u in range(4):
                    i = i4 * 4 + u
                    a0 = rows[i, pl.ds(0, 16)]
                    a1 = rows[i, pl.ds(16, 16)]
                    for t in range(1, L):
                        a0 = a0 + rows[t * _RW + i, pl.ds(0, 16)]
                        a1 = a1 + rows[t * _RW + i, pl.ds(16, 16)]
                    pool[i, pl.ds(0, 16)] = a0
                    pool[i, pl.ds(16, 16)] = a1
                return carry

            lax.fori_loop(0, _RW // 4, body, 0)

        name_gather.wait()
        reduce_rows(rows_l)
        pltpu.sync_copy(pool, ea.at[0, pl.ds(base, _RW), :])

        gen_gather.wait()
        reduce_rows(rows_l2)
        pltpu.sync_copy(pool, ea.at[1, pl.ds(base, _RW), :])

    return sc_pool


def _make_sc_big():
    mesh = plsc.VectorSubcoreMesh(
        core_axis_name="c", subcore_axis_name="s",
        num_cores=_NC, num_subcores=_NS)

    @functools.partial(
        pl.kernel,
        out_type=jax.ShapeDtypeStruct((3, B, D), jnp.float32),
        mesh=mesh,
        scratch_types=[
            pltpu.VMEM((144,), jnp.int32),
            pltpu.VMEM((2, _CH, D, 128), jnp.float32),
            pltpu.VMEM((_CH, D), jnp.float32),
            pltpu.SemaphoreType.DMA,
            pltpu.SemaphoreType.DMA,
        ],
        compiler_params=pltpu.CompilerParams(needs_layout_passes=False),
    )
    def sc_big(track_i, artist_i, album_i, t_track, t_artist, t_album,
               eb, idx_pad, blk, pool8, semA, semB):
        wid = lax.axis_index("s") * _NC + lax.axis_index("c")
        base = wid * _RW
        c_lo = lax.iota(jnp.int32, 16)
        sems = (semA, semB)

        def fire(th, ch, b):
            iv = idx_pad[pl.ds(ch * _CH, 16)]
            for k in range(_CH):
                joff = pl.multiple_of((iv[k] >> 7) * 128, 128)
                pltpu.async_copy(
                    th.at[:, pl.ds(joff, 128)], blk.at[b, k], sems[b])

        def drain(th, b):
            for k in range(_CH):
                pltpu.make_async_copy(
                    th.at[:, pl.ds(0, 128)], blk.at[b, k], sems[b]).wait()

        def extract(f, ch, b):
            iv = idx_pad[pl.ds(ch * _CH, 16)]
            bb = jnp.full((16,), b, jnp.int32)
            for k in range(_CH):
                col = jnp.full((16,), iv[k] & 127, jnp.int32)
                kk = jnp.full((16,), k, jnp.int32)
                pool8[k, pl.ds(0, 16)] = plsc.load_gather(
                    blk, [bb, kk, c_lo, col])
                pool8[k, pl.ds(16, 16)] = plsc.load_gather(
                    blk, [bb, kk, c_lo + 16, col])
            pltpu.sync_copy(
                pool8, eb.at[f, pl.ds(base + ch * _CH, _CH), :])

        for f, ih, th in ((0, track_i, t_track), (1, artist_i, t_artist),
                          (2, album_i, t_album)):
            pltpu.sync_copy(ih.at[pl.ds(base, _RW)],
                            idx_pad.at[pl.ds(0, _RW)])
            fire(th, 0, 0)
            fire(th, 1, 1)

            def pair_body(j, carry, th=th, f=f):
                for b in range(2):
                    ch = j * 2 + b
                    drain(th, b)
                    extract(f, ch, b)
                    fire(th, ch + 2, b)
                return carry

            lax.fori_loop(0, _NCH // 2 - 1, pair_body, 0)
            for b in range(2):
                drain(th, b)
                extract(f, _NCH - 2 + b, b)

    return sc_big


_sc_pool_cached = functools.cache(_make_sc_pool)
_sc_big_cached = functools.cache(_make_sc_big)

_R = 512


def _mlp_body(ea_ref, eb_ref, n_ref, gidx_ref, row0_ref, w1a_ref, w1b_ref,
              w1n_ref, b1_ref, w2_ref, b2_ref, w3_ref, b3_ref, g_ref,
              bt_ref, out_ref):
    f32 = jnp.float32
    h = jnp.dot(n_ref[...], w1n_ref[...], preferred_element_type=f32)
    h = h + jnp.dot(ea_ref[0], w1a_ref[0], preferred_element_type=f32)
    n0 = jnp.sum((gidx_ref[...] == 0).astype(f32), axis=1, keepdims=True)
    eg = (ea_ref[1] - n0 * row0_ref[...]) / jnp.maximum(8.0 - n0, 1.0)
    h = h + jnp.dot(eg, w1a_ref[1], preferred_element_type=f32)
    for f in range(2, 5):
        h = h + jnp.dot(ea_ref[f], w1a_ref[f], preferred_element_type=f32)
    for f in range(3):
        h = h + jnp.dot(eb_ref[f], w1b_ref[f], preferred_element_type=f32)
    h = jnp.maximum(h + b1_ref[...], 0.0)
    h = jnp.dot(h, w2_ref[...], preferred_element_type=f32) + b2_ref[...]
    h = jnp.maximum(h, 0.0)
    h = jnp.dot(h, w3_ref[...], preferred_element_type=f32) + b3_ref[...]
    mu = jnp.mean(h, axis=1, keepdims=True)
    d = h - mu
    var = jnp.mean(d * d, axis=1, keepdims=True)
    out_ref[...] = g_ref[...] * d / jnp.sqrt(var + 1e-3) + bt_ref[...]


def _mlp_call(ea, eb, nmat, gidx, row0, w1a, w1b, w1n, b1, w2, b2, w3, b3,
              gamma, beta):
    full = lambda shape: pl.BlockSpec(shape, lambda i: (0, 0))
    return pl.pallas_call(
        _mlp_body,
        grid=(B // _R,),
        in_specs=[
            pl.BlockSpec((5, _R, D), lambda i: (0, i, 0)),
            pl.BlockSpec((3, _R, D), lambda i: (0, i, 0)),
            pl.BlockSpec((_R, 16), lambda i: (i, 0)),
            pl.BlockSpec((_R, L), lambda i: (i, 0)),
            full((1, D)),
            pl.BlockSpec((5, D, 512), lambda i: (0, 0, 0)),
            pl.BlockSpec((3, D, 512), lambda i: (0, 0, 0)),
            full((16, 512)), full((1, 512)),
            full((512, 256)), full((1, 256)),
            full((256, 128)), full((1, 128)),
            full((1, 128)), full((1, 128)),
        ],
        out_specs=pl.BlockSpec((_R, 128), lambda i: (i, 0)),
        out_shape=jax.ShapeDtypeStruct((B, 128), jnp.float32),
    )(ea, eb, nmat, gidx, row0, w1a, w1b, w1n, b1, w2, b2, w3, b3, gamma,
      beta)


def kernel(track_uri_can, track_name_can, artist_uri_can, album_uri_can,
           artist_genres_can, track_key_can, track_mode_can,
           time_signature_can, duration_ms_can, track_pop_can,
           artist_pop_can, artist_followers_can, track_danceability_can,
           track_energy_can, track_loudness_can, track_speechiness_can,
           track_acousticness_can, track_instrumentalness_can,
           track_liveness_can, track_valence_can, track_tempo_can,
           track_uri_table, track_name_table, artist_uri_table,
           album_uri_table, artist_genres_table, track_key_table,
           track_mode_table, time_signature_table, W1, b1, W2, b2, W3, b3,
           gamma, beta):
    i32 = jnp.int32
    eb = _sc_big_cached()(
        track_uri_can.astype(i32),
        artist_uri_can.astype(i32),
        album_uri_can.astype(i32),
        track_uri_table.T, artist_uri_table.T, album_uri_table.T)
    key_gate, eb = lax.optimization_barrier(
        (track_key_can.astype(i32), eb))
    ea = _sc_pool_cached()(
        track_name_can.astype(i32).T,
        artist_genres_can.astype(i32).T,
        key_gate,
        track_mode_can.astype(i32),
        time_signature_can.astype(i32),
        track_name_table, artist_genres_table, track_key_table,
        track_mode_table, time_signature_table)

    z = jnp.zeros_like(duration_ms_can)
    nmat = jnp.stack(
        [duration_ms_can, track_pop_can, artist_pop_can,
         artist_followers_can, track_danceability_can, track_energy_can,
         track_loudness_can, track_speechiness_can, track_acousticness_can,
         track_instrumentalness_can, track_liveness_can, track_valence_can,
         track_tempo_can, z, z, z], axis=1)

    w1a = jnp.stack(
        [W1[32:32 + D] * (1.0 / L)] +
        [W1[s:s + D] for s in _POOL_ROW_STARTS[1:]], axis=0)
    w1b = jnp.stack([W1[s:s + D] for s in _BIG_ROW_STARTS], axis=0)
    w1n = jnp.concatenate(
        [W1[jnp.array(_NUM_ROWS)], jnp.zeros((3, W1.shape[1]), W1.dtype)],
        axis=0)

    return _mlp_call(ea, eb, nmat, artist_genres_can.astype(i32),
                     artist_genres_table[0:1], w1a, w1b, w1n,
                     b1.reshape(1, -1), W2, b2.reshape(1, -1), W3,
                     b3.reshape(1, -1), gamma.reshape(1, -1),
                     beta.reshape(1, -1))
